# Initial kernel scaffold; baseline (speedup 1.0000x reference)
import jax
import jax.numpy as jnp
from jax.experimental import pallas as pl


def kernel(in_values, active_out_indices, W, bias, proj, buckets):
    raise NotImplementedError("write your pallas kernel here")


# speedup vs baseline: 15.9744x; 15.9744x over previous
import functools

import jax
import jax.numpy as jnp
from jax import lax
from jax.experimental import pallas as pl
from jax.experimental.pallas import tpu as pltpu
from jax.experimental.pallas import tpu_sc as plsc


def _hash_body(x_ref, proj_ref, row_ref):
    s = lax.dot_general(
        x_ref[...], proj_ref[...], (((1,), (0,)), ((), ())),
        preferred_element_type=jnp.float32, precision=lax.Precision.DEFAULT)
    LK = proj_ref.shape[1]
    L = row_ref.shape[1]
    K = LK // L
    bits = (s > 0).astype(jnp.float32)
    jj = lax.broadcasted_iota(jnp.int32, (LK, L), 0)
    ll = lax.broadcasted_iota(jnp.int32, (LK, L), 1)
    amt = jj - K * ll
    sel = (amt >= 0) & (amt < K)
    M = jnp.where(sel, jnp.int32(1) << jnp.clip(amt, 0, K - 1), 0)
    h = lax.dot_general(
        bits, M.astype(jnp.float32), (((1,), (0,)), ((), ())),
        preferred_element_type=jnp.float32, precision=lax.Precision.HIGHEST)
    lrow = lax.broadcasted_iota(jnp.int32, h.shape, 1)
    row_ref[...] = h.astype(jnp.int32) + jnp.int32(2 ** K) * lrow


def _logits_body(x_ref, w_ref, b_ref, y_ref):
    y = lax.dot_general(
        x_ref[...], w_ref[...], (((1,), (1,)), ((), ())),
        preferred_element_type=jnp.float32, precision=lax.Precision.HIGHEST)
    y_ref[...] = y + b_ref[...]


def _sc_body(out_dim, spw, row_hbm, bkt_hbm, y_hbm, act_hbm, val_hbm,
             idxs_v, cand_v, absidx_v, vals_v, sem_a, sem_b):
    nc = 2
    wid = lax.axis_index("s") * nc + lax.axis_index("c")
    words = spw * 8
    base_w = wid * words
    for j in range(words // 128):
        pltpu.sync_copy(row_hbm.at[pl.ds(base_w + j * 128, 128)],
                        idxs_v.at[j])
    for j in range(words // 128):
        pltpu.async_copy(bkt_hbm.at[idxs_v.at[j]],
                         cand_v.at[pl.ds(j * 128, 128)], sem_a).wait()
    pltpu.sync_copy(cand_v, act_hbm.at[wid])

    base_s = wid * spw
    rows_per_sample = (spw * 512) // 128 // spw

    def absbody(r, carry):
        boff = (base_s + r // rows_per_sample) * out_dim
        for h in range(8):
            sr = 2 * r + (h // 4)
            cc = (h % 4) * 16
            absidx_v[r, pl.ds(h * 16, 16)] = cand_v[sr, pl.ds(cc, 16)] + boff
        return carry

    lax.fori_loop(0, spw * rows_per_sample, absbody, 0)

    def gbody(g, carry):
        handles = []
        for k in range(8):
            r = g * 8 + k
            handles.append(
                pltpu.async_copy(y_hbm.at[absidx_v.at[r]], vals_v.at[r],
                                 sem_b))
        for hnd in handles:
            hnd.wait()
        return carry

    lax.fori_loop(0, (spw * rows_per_sample) // 8, gbody, 0)
    pltpu.sync_copy(vals_v, val_hbm.at[wid])


def kernel(in_values, active_out_indices, W, bias, proj, buckets):
    B, D = in_values.shape
    out_dim = W.shape[0]
    Lt, nbk, bs = buckets.shape
    n_active = active_out_indices.shape[1]

    rowflat = pl.pallas_call(
        _hash_body,
        out_shape=jax.ShapeDtypeStruct((B, Lt), jnp.int32),
    )(in_values, proj)

    bn = 2048
    grid = (pl.cdiv(out_dim, bn),)
    Y = pl.pallas_call(
        _logits_body,
        grid=grid,
        in_specs=[
            pl.BlockSpec((B, D), lambda i: (0, 0)),
            pl.BlockSpec((bn, D), lambda i: (i, 0)),
            pl.BlockSpec((1, bn), lambda i: (0, i)),
        ],
        out_specs=pl.BlockSpec((B, bn), lambda i: (0, i)),
        out_shape=jax.ShapeDtypeStruct((B, out_dim), jnp.float32),
    )(in_values, W, bias.reshape(1, out_dim))

    info = plsc.get_sparse_core_info()
    nw = info.num_cores * info.num_subcores
    spw = B // nw
    words = spw * Lt
    vrows = (spw * Lt * bs) // 128

    mesh = plsc.VectorSubcoreMesh(core_axis_name="c", subcore_axis_name="s")
    sc = pl.kernel(
        functools.partial(_sc_body, out_dim, spw),
        out_type=[
            jax.ShapeDtypeStruct((nw, words, bs), jnp.int32),
            jax.ShapeDtypeStruct((nw, vrows, 128), jnp.float32),
        ],
        mesh=mesh,
        compiler_params=pltpu.CompilerParams(use_tc_tiling_on_sc=False),
        scratch_types=[
            pltpu.VMEM((words // 128, 128), jnp.int32),
            pltpu.VMEM((words, bs), jnp.int32),
            pltpu.VMEM((vrows, 128), jnp.int32),
            pltpu.VMEM((vrows, 128), jnp.float32),
            pltpu.SemaphoreType.DMA,
            pltpu.SemaphoreType.DMA,
        ],
    )
    act, vals = sc(rowflat.reshape(B * Lt),
                   buckets.reshape(Lt * nbk, bs).astype(jnp.int32),
                   Y.reshape(B * out_dim))

    active = act.reshape(B, n_active).astype(jnp.int64)
    out = vals.reshape(B, n_active)
    return (out, active)


# speedup vs baseline: 19.6072x; 1.2274x over previous
import functools

import jax
import jax.numpy as jnp
from jax import lax
from jax.experimental import pallas as pl
from jax.experimental.pallas import tpu as pltpu
from jax.experimental.pallas import tpu_sc as plsc


def _hash_body(x_ref, proj_ref, row_ref):
    s = lax.dot_general(
        x_ref[...], proj_ref[...], (((1,), (0,)), ((), ())),
        preferred_element_type=jnp.float32, precision=lax.Precision.DEFAULT)
    LK = proj_ref.shape[1]
    L = row_ref.shape[1]
    K = LK // L
    bits = (s > 0).astype(jnp.float32)
    jj = lax.broadcasted_iota(jnp.int32, (LK, L), 0)
    ll = lax.broadcasted_iota(jnp.int32, (LK, L), 1)
    amt = jj - K * ll
    sel = (amt >= 0) & (amt < K)
    M = jnp.where(sel, jnp.int32(1) << jnp.clip(amt, 0, K - 1), 0)
    h = lax.dot_general(
        bits, M.astype(jnp.float32), (((1,), (0,)), ((), ())),
        preferred_element_type=jnp.float32, precision=lax.Precision.HIGHEST)
    lrow = lax.broadcasted_iota(jnp.int32, h.shape, 1)
    row_ref[...] = h.astype(jnp.int32) + jnp.int32(2 ** K) * lrow


def _logits_body(x_ref, w_ref, b_ref, y_ref):
    y = lax.dot_general(
        x_ref[...], w_ref[...], (((1,), (1,)), ((), ())),
        preferred_element_type=jnp.float32, precision=lax.Precision.DEFAULT)
    y_ref[...] = y + b_ref[...]


def _sc_body(out_dim, spw, row_hbm, bkt_hbm, y_hbm, act_hbm, val_hbm,
             idxs_v, cand_v, absidx_v, vals_v, sem_a, sem_b):
    nc = 2
    wid = lax.axis_index("s") * nc + lax.axis_index("c")
    words = spw * 8
    base_w = wid * words
    for j in range(words // 128):
        pltpu.sync_copy(row_hbm.at[pl.ds(base_w + j * 128, 128)],
                        idxs_v.at[j])
    for j in range(words // 128):
        pltpu.async_copy(bkt_hbm.at[idxs_v.at[j]],
                         cand_v.at[pl.ds(j * 128, 128)], sem_a).wait()
    pltpu.sync_copy(cand_v, act_hbm.at[wid])

    base_s = wid * spw
    rows_per_sample = (spw * 512) // 128 // spw

    def absbody(r, carry):
        boff = (base_s + r // rows_per_sample) * out_dim
        for h in range(8):
            sr = 2 * r + (h // 4)
            cc = (h % 4) * 16
            absidx_v[r, pl.ds(h * 16, 16)] = cand_v[sr, pl.ds(cc, 16)] + boff
        return carry

    lax.fori_loop(0, spw * rows_per_sample, absbody, 0)

    def gbody(g, carry):
        handles = []
        for k in range(8):
            r = g * 8 + k
            handles.append(
                pltpu.async_copy(y_hbm.at[absidx_v.at[r]], vals_v.at[r],
                                 sem_b))
        for hnd in handles:
            hnd.wait()
        return carry

    lax.fori_loop(0, (spw * rows_per_sample) // 8, gbody, 0)
    pltpu.sync_copy(vals_v, val_hbm.at[wid])


def kernel(in_values, active_out_indices, W, bias, proj, buckets):
    B, D = in_values.shape
    out_dim = W.shape[0]
    Lt, nbk, bs = buckets.shape
    n_active = active_out_indices.shape[1]

    rowflat = pl.pallas_call(
        _hash_body,
        out_shape=jax.ShapeDtypeStruct((B, Lt), jnp.int32),
    )(in_values, proj)

    bn = 2048
    grid = (pl.cdiv(out_dim, bn),)
    Y = pl.pallas_call(
        _logits_body,
        grid=grid,
        in_specs=[
            pl.BlockSpec((B, D), lambda i: (0, 0)),
            pl.BlockSpec((bn, D), lambda i: (i, 0)),
            pl.BlockSpec((1, bn), lambda i: (0, i)),
        ],
        out_specs=pl.BlockSpec((B, bn), lambda i: (0, i)),
        out_shape=jax.ShapeDtypeStruct((B, out_dim), jnp.float32),
    )(in_values, W, bias.reshape(1, out_dim))

    info = plsc.get_sparse_core_info()
    nw = info.num_cores * info.num_subcores
    spw = B // nw
    words = spw * Lt
    vrows = (spw * Lt * bs) // 128

    mesh = plsc.VectorSubcoreMesh(core_axis_name="c", subcore_axis_name="s")
    sc = pl.kernel(
        functools.partial(_sc_body, out_dim, spw),
        out_type=[
            jax.ShapeDtypeStruct((nw, words, bs), jnp.int32),
            jax.ShapeDtypeStruct((nw, vrows, 128), jnp.float32),
        ],
        mesh=mesh,
        compiler_params=pltpu.CompilerParams(use_tc_tiling_on_sc=False),
        scratch_types=[
            pltpu.VMEM((words // 128, 128), jnp.int32),
            pltpu.VMEM((words, bs), jnp.int32),
            pltpu.VMEM((vrows, 128), jnp.int32),
            pltpu.VMEM((vrows, 128), jnp.float32),
            pltpu.SemaphoreType.DMA,
            pltpu.SemaphoreType.DMA,
        ],
    )
    act, vals = sc(rowflat.reshape(B * Lt),
                   buckets.reshape(Lt * nbk, bs).astype(jnp.int32),
                   Y.reshape(B * out_dim))

    active = act.reshape(B, n_active).astype(jnp.int64)
    out = vals.reshape(B, n_active)
    return (out, active)


# speedup vs baseline: 66.2560x; 3.3792x over previous
import functools

import jax
import jax.numpy as jnp
from jax import lax
from jax.experimental import pallas as pl
from jax.experimental.pallas import tpu as pltpu
from jax.experimental.pallas import tpu_sc as plsc


def _hash_body(x_ref, proj_ref, row_ref):
    s = lax.dot_general(
        x_ref[...], proj_ref[...], (((1,), (0,)), ((), ())),
        preferred_element_type=jnp.float32, precision=lax.Precision.DEFAULT)
    LK = proj_ref.shape[1]
    L = row_ref.shape[1]
    K = LK // L
    bits = (s > 0).astype(jnp.float32)
    jj = lax.broadcasted_iota(jnp.int32, (LK, L), 0)
    ll = lax.broadcasted_iota(jnp.int32, (LK, L), 1)
    amt = jj - K * ll
    sel = (amt >= 0) & (amt < K)
    M = jnp.where(sel, jnp.int32(1) << jnp.clip(amt, 0, K - 1), 0)
    h = lax.dot_general(
        bits, M.astype(jnp.float32), (((1,), (0,)), ((), ())),
        preferred_element_type=jnp.float32, precision=lax.Precision.HIGHEST)
    lrow = lax.broadcasted_iota(jnp.int32, h.shape, 1)
    row_ref[...] = h.astype(jnp.int32) + jnp.int32(2 ** K) * lrow


def _logits_body(x_ref, w_ref, b_ref, y_ref):
    y = lax.dot_general(
        x_ref[...], w_ref[...], (((1,), (1,)), ((), ())),
        preferred_element_type=jnp.float32, precision=lax.Precision.DEFAULT)
    y = y + b_ref[...]
    y_ref[...] = y.reshape(y_ref.shape)


def _sc_body(out_dim, spw, row_hbm, bkt_hbm, y_hbm, act_hbm, val_hbm,
             idxs_v, cand_v, absidx_v, vals_v, sem_a, sem_b):
    nc = 2
    wid = lax.axis_index("s") * nc + lax.axis_index("c")
    words = spw * 8
    base_w = wid * words
    for j in range(words // 128):
        pltpu.sync_copy(row_hbm.at[pl.ds(base_w + j * 128, 128)],
                        idxs_v.at[j])
    for j in range(words // 128):
        pltpu.async_copy(bkt_hbm.at[idxs_v.at[j]],
                         cand_v.at[pl.ds(j * 128, 128)], sem_a).wait()
    pltpu.sync_copy(cand_v, act_hbm.at[wid])

    base_s = wid * spw
    rows_per_sample = (spw * 512) // 128 // spw

    def absbody(r, carry):
        boff = (base_s + r // rows_per_sample) * out_dim
        for h in range(8):
            sr = 2 * r + (h // 4)
            cc = (h % 4) * 16
            absidx_v[r, pl.ds(h * 16, 16)] = cand_v[sr, pl.ds(cc, 16)] + boff
        return carry

    lax.fori_loop(0, spw * rows_per_sample, absbody, 0)

    def gbody(g, carry):
        handles = []
        for k in range(8):
            r = g * 8 + k
            handles.append(
                pltpu.async_copy(y_hbm.at[absidx_v.at[r]], vals_v.at[r],
                                 sem_b))
        for hnd in handles:
            hnd.wait()
        return carry

    lax.fori_loop(0, (spw * rows_per_sample) // 8, gbody, 0)
    pltpu.sync_copy(vals_v, val_hbm.at[wid])


def kernel(in_values, active_out_indices, W, bias, proj, buckets):
    B, D = in_values.shape
    out_dim = W.shape[0]
    Lt, nbk, bs = buckets.shape
    n_active = active_out_indices.shape[1]

    rowflat = pl.pallas_call(
        _hash_body,
        out_shape=jax.ShapeDtypeStruct((B, Lt), jnp.int32),
    )(in_values, proj)

    bn = 2048
    n_blk = pl.cdiv(out_dim, bn)
    out_pad = n_blk * bn
    Y = pl.pallas_call(
        _logits_body,
        grid=(n_blk,),
        in_specs=[
            pl.BlockSpec((B, D), lambda i: (0, 0)),
            pl.BlockSpec((bn, D), lambda i: (i, 0)),
            pl.BlockSpec((1, bn), lambda i: (0, i)),
        ],
        out_specs=pl.BlockSpec((B, bn // 128, 128), lambda i: (0, i, 0)),
        out_shape=jax.ShapeDtypeStruct((B, out_pad // 128, 128), jnp.float32),
    )(in_values, W, bias.reshape(1, out_dim))

    info = plsc.get_sparse_core_info()
    nw = info.num_cores * info.num_subcores
    spw = B // nw
    words = spw * Lt
    vrows = (spw * Lt * bs) // 128

    mesh = plsc.VectorSubcoreMesh(core_axis_name="c", subcore_axis_name="s")
    sc = pl.kernel(
        functools.partial(_sc_body, out_pad, spw),
        out_type=[
            jax.ShapeDtypeStruct((nw, words, bs), jnp.int32),
            jax.ShapeDtypeStruct((nw, vrows, 128), jnp.float32),
        ],
        mesh=mesh,
        compiler_params=pltpu.CompilerParams(use_tc_tiling_on_sc=False),
        scratch_types=[
            pltpu.VMEM((words // 128, 128), jnp.int32),
            pltpu.VMEM((words, bs), jnp.int32),
            pltpu.VMEM((vrows, 128), jnp.int32),
            pltpu.VMEM((vrows, 128), jnp.float32),
            pltpu.SemaphoreType.DMA,
            pltpu.SemaphoreType.DMA,
        ],
    )
    act, vals = sc(rowflat.reshape(B * Lt),
                   buckets.reshape(Lt * nbk, bs).astype(jnp.int32),
                   Y.reshape(B * out_pad))

    active = act.reshape(B, n_active).astype(jnp.int64)
    out = vals.reshape(B, n_active)
    return (out, active)


# speedup vs baseline: 69.3951x; 1.0474x over previous
import functools

import jax
import jax.numpy as jnp
from jax import lax
from jax.experimental import pallas as pl
from jax.experimental.pallas import tpu as pltpu
from jax.experimental.pallas import tpu_sc as plsc


def _hash_body(x_ref, proj_ref, row_ref):
    s = lax.dot_general(
        x_ref[...], proj_ref[...], (((1,), (0,)), ((), ())),
        preferred_element_type=jnp.float32, precision=lax.Precision.DEFAULT)
    LK = proj_ref.shape[1]
    L = row_ref.shape[1]
    K = LK // L
    bits = (s > 0).astype(jnp.float32)
    jj = lax.broadcasted_iota(jnp.int32, (LK, L), 0)
    ll = lax.broadcasted_iota(jnp.int32, (LK, L), 1)
    amt = jj - K * ll
    sel = (amt >= 0) & (amt < K)
    M = jnp.where(sel, jnp.int32(1) << jnp.clip(amt, 0, K - 1), 0)
    h = lax.dot_general(
        bits, M.astype(jnp.float32), (((1,), (0,)), ((), ())),
        preferred_element_type=jnp.float32, precision=lax.Precision.HIGHEST)
    lrow = lax.broadcasted_iota(jnp.int32, h.shape, 1)
    row_ref[...] = h.astype(jnp.int32) + jnp.int32(2 ** K) * lrow


def _logits_body(x_ref, w_ref, b_ref, y_ref):
    y = lax.dot_general(
        x_ref[...], w_ref[...], (((1,), (1,)), ((), ())),
        preferred_element_type=jnp.float32, precision=lax.Precision.DEFAULT)
    y = y + b_ref[...]
    y_ref[...] = y.reshape(y_ref.shape)


def _sc_body(out_dim, spw, row_hbm, bkt_hbm, y_hbm, act_hbm, val_hbm,
             idxs_v, cand_v, absidx_v, vals_v, sem_a, sem_b):
    nc = 2
    wid = lax.axis_index("s") * nc + lax.axis_index("c")
    words = spw * 8
    base_w = wid * words
    n_seg = words // 128
    for j in range(n_seg):
        pltpu.sync_copy(row_hbm.at[pl.ds(base_w + j * 128, 128)],
                        idxs_v.at[j])
    bkt_handles = [
        pltpu.async_copy(bkt_hbm.at[idxs_v.at[j]],
                         cand_v.at[pl.ds(j * 128, 128)], sem_a)
        for j in range(n_seg)
    ]
    for hnd in bkt_handles:
        hnd.wait()
    act_handle = pltpu.async_copy(cand_v, act_hbm.at[wid], sem_a)

    base_s = wid * spw
    vrows = (spw * 512) // 128
    rows_per_sample = vrows // spw

    def absbody(r, carry):
        boff = (base_s + r // rows_per_sample) * out_dim
        for h in range(8):
            sr = 2 * r + (h // 4)
            cc = (h % 4) * 16
            absidx_v[r, pl.ds(h * 16, 16)] = cand_v[sr, pl.ds(cc, 16)] + boff
        return carry

    lax.fori_loop(0, vrows, absbody, 0)

    def fire(r, carry):
        pltpu.async_copy(y_hbm.at[absidx_v.at[r]], vals_v.at[r], sem_b)
        return carry

    lax.fori_loop(0, vrows, fire, 0)

    def drain(r, carry):
        pltpu.make_async_copy(y_hbm.at[absidx_v.at[r]], vals_v.at[r],
                              sem_b).wait()
        return carry

    lax.fori_loop(0, vrows, drain, 0)
    pltpu.sync_copy(vals_v, val_hbm.at[wid])
    act_handle.wait()


def kernel(in_values, active_out_indices, W, bias, proj, buckets):
    B, D = in_values.shape
    out_dim = W.shape[0]
    Lt, nbk, bs = buckets.shape
    n_active = active_out_indices.shape[1]

    rowflat = pl.pallas_call(
        _hash_body,
        out_shape=jax.ShapeDtypeStruct((B, Lt), jnp.int32),
    )(in_values, proj)

    bn = 2048
    n_blk = pl.cdiv(out_dim, bn)
    out_pad = n_blk * bn
    Y = pl.pallas_call(
        _logits_body,
        grid=(n_blk,),
        in_specs=[
            pl.BlockSpec((B, D), lambda i: (0, 0)),
            pl.BlockSpec((bn, D), lambda i: (i, 0)),
            pl.BlockSpec((1, bn), lambda i: (0, i)),
        ],
        out_specs=pl.BlockSpec((B, bn // 128, 128), lambda i: (0, i, 0)),
        out_shape=jax.ShapeDtypeStruct((B, out_pad // 128, 128), jnp.float32),
    )(in_values, W, bias.reshape(1, out_dim))

    info = plsc.get_sparse_core_info()
    nw = info.num_cores * info.num_subcores
    spw = B // nw
    words = spw * Lt
    vrows = (spw * Lt * bs) // 128

    mesh = plsc.VectorSubcoreMesh(core_axis_name="c", subcore_axis_name="s")
    sc = pl.kernel(
        functools.partial(_sc_body, out_pad, spw),
        out_type=[
            jax.ShapeDtypeStruct((nw, words, bs), jnp.int32),
            jax.ShapeDtypeStruct((nw, vrows, 128), jnp.float32),
        ],
        mesh=mesh,
        compiler_params=pltpu.CompilerParams(use_tc_tiling_on_sc=False),
        scratch_types=[
            pltpu.VMEM((words // 128, 128), jnp.int32),
            pltpu.VMEM((words, bs), jnp.int32),
            pltpu.VMEM((vrows, 128), jnp.int32),
            pltpu.VMEM((vrows, 128), jnp.float32),
            pltpu.SemaphoreType.DMA,
            pltpu.SemaphoreType.DMA,
        ],
    )
    act, vals = sc(rowflat.reshape(B * Lt),
                   buckets.reshape(Lt * nbk, bs).astype(jnp.int32),
                   Y.reshape(B * out_pad))

    active = act.reshape(B, n_active).astype(jnp.int64)
    out = vals.reshape(B, n_active)
    return (out, active)


# speedup vs baseline: 84.0301x; 1.2109x over previous
import functools

import jax
import jax.numpy as jnp
from jax import lax
from jax.experimental import pallas as pl
from jax.experimental.pallas import tpu as pltpu
from jax.experimental.pallas import tpu_sc as plsc


def _hash_body(x_ref, proj_ref, row_ref):
    s = lax.dot_general(
        x_ref[...], proj_ref[...], (((1,), (0,)), ((), ())),
        preferred_element_type=jnp.float32, precision=lax.Precision.DEFAULT)
    LK = proj_ref.shape[1]
    L = row_ref.shape[1]
    K = LK // L
    bits = (s > 0).astype(jnp.float32)
    jj = lax.broadcasted_iota(jnp.int32, (LK, L), 0)
    ll = lax.broadcasted_iota(jnp.int32, (LK, L), 1)
    amt = jj - K * ll
    sel = (amt >= 0) & (amt < K)
    M = jnp.where(sel, jnp.int32(1) << jnp.clip(amt, 0, K - 1), 0)
    h = lax.dot_general(
        bits, M.astype(jnp.float32), (((1,), (0,)), ((), ())),
        preferred_element_type=jnp.float32, precision=lax.Precision.HIGHEST)
    lrow = lax.broadcasted_iota(jnp.int32, h.shape, 1)
    row_ref[...] = h.astype(jnp.int32) + jnp.int32(2 ** K) * lrow


def _logits_body(x_ref, w_ref, b_ref, y_ref):
    y = lax.dot_general(
        x_ref[...], w_ref[...], (((1,), (1,)), ((), ())),
        preferred_element_type=jnp.float32, precision=lax.Precision.DEFAULT)
    y = y + b_ref[...]
    bn = y.shape[1]
    u = lax.bitcast_convert_type(y, jnp.uint32)
    packed = (u[:, :bn // 2] >> 16) | (u[:, bn // 2:] & jnp.uint32(0xFFFF0000))
    y_ref[...] = packed.reshape(y_ref.shape)


def _sc_body(out_half, spw, row_hbm, bkt_hbm, y_hbm, act_hbm, val_hbm,
             idxs_v, cand_v, absidx_v, half_v, vals_v, vals_f, sem_a, sem_b):
    nc = 2
    wid = lax.axis_index("s") * nc + lax.axis_index("c")
    words = spw * 8
    base_w = wid * words
    n_seg = words // 128
    for j in range(n_seg):
        pltpu.sync_copy(row_hbm.at[pl.ds(base_w + j * 128, 128)],
                        idxs_v.at[j])
    bkt_handles = [
        pltpu.async_copy(bkt_hbm.at[idxs_v.at[j]],
                         cand_v.at[pl.ds(j * 128, 128)], sem_a)
        for j in range(n_seg)
    ]
    for hnd in bkt_handles:
        hnd.wait()
    act_handle = pltpu.async_copy(cand_v, act_hbm.at[wid], sem_a)

    base_s = wid * spw
    vrows = (spw * 512) // 128
    rows_per_sample = vrows // spw

    def absbody(r, carry):
        boff = (base_s + r // rows_per_sample) * out_half
        for h in range(8):
            sr = 2 * r + (h // 4)
            cc = (h % 4) * 16
            c = cand_v[sr, pl.ds(cc, 16)]
            absidx_v[r, pl.ds(h * 16, 16)] = (
                boff + ((c >> 11) << 10) + (c & 1023))
            cu = plsc.bitcast(c, jnp.uint32)
            half_v[r, pl.ds(h * 16, 16)] = (cu >> 10) & jnp.uint32(1)
        return carry

    lax.fori_loop(0, vrows, absbody, 0)

    def fire(r, carry):
        pltpu.async_copy(y_hbm.at[absidx_v.at[r]], vals_v.at[r], sem_b)
        return carry

    lax.fori_loop(0, vrows, fire, 0)

    def drain(r, carry):
        pltpu.make_async_copy(y_hbm.at[absidx_v.at[r]], vals_v.at[r],
                              sem_b).wait()
        for h in range(8):
            sl = pl.ds(h * 16, 16)
            w = vals_v[r, sl]
            sh = jnp.uint32(16) - (half_v[r, sl] << 4)
            t = (w << sh) & jnp.uint32(0xFFFF0000)
            vals_f[r, sl] = plsc.bitcast(t, jnp.float32)
        return carry

    lax.fori_loop(0, vrows, drain, 0)
    pltpu.sync_copy(vals_f, val_hbm.at[wid])
    act_handle.wait()


def kernel(in_values, active_out_indices, W, bias, proj, buckets):
    B, D = in_values.shape
    out_dim = W.shape[0]
    Lt, nbk, bs = buckets.shape
    n_active = active_out_indices.shape[1]

    rowflat = pl.pallas_call(
        _hash_body,
        out_shape=jax.ShapeDtypeStruct((B, Lt), jnp.int32),
    )(in_values, proj)

    bn = 2048
    n_blk = pl.cdiv(out_dim, bn)
    out_pad = n_blk * bn
    Y = pl.pallas_call(
        _logits_body,
        grid=(n_blk,),
        in_specs=[
            pl.BlockSpec((B, D), lambda i: (0, 0)),
            pl.BlockSpec((bn, D), lambda i: (i, 0)),
            pl.BlockSpec((1, bn), lambda i: (0, i)),
        ],
        out_specs=pl.BlockSpec((B, bn // 256, 128), lambda i: (0, i, 0)),
        out_shape=jax.ShapeDtypeStruct((B, out_pad // 256, 128), jnp.uint32),
    )(in_values, W, bias.reshape(1, out_dim))

    info = plsc.get_sparse_core_info()
    nw = info.num_cores * info.num_subcores
    spw = B // nw
    words = spw * Lt
    vrows = (spw * Lt * bs) // 128

    mesh = plsc.VectorSubcoreMesh(core_axis_name="c", subcore_axis_name="s")
    sc = pl.kernel(
        functools.partial(_sc_body, out_pad // 2, spw),
        out_type=[
            jax.ShapeDtypeStruct((nw, words, bs), jnp.int32),
            jax.ShapeDtypeStruct((nw, vrows, 128), jnp.float32),
        ],
        mesh=mesh,
        compiler_params=pltpu.CompilerParams(use_tc_tiling_on_sc=False,
                                             needs_layout_passes=False),
        scratch_types=[
            pltpu.VMEM((words // 128, 128), jnp.int32),
            pltpu.VMEM((words, bs), jnp.int32),
            pltpu.VMEM((vrows, 128), jnp.int32),
            pltpu.VMEM((vrows, 128), jnp.uint32),
            pltpu.VMEM((vrows, 128), jnp.uint32),
            pltpu.VMEM((vrows, 128), jnp.float32),
            pltpu.SemaphoreType.DMA,
            pltpu.SemaphoreType.DMA,
        ],
    )
    act, vals = sc(rowflat.reshape(B * Lt),
                   buckets.reshape(Lt * nbk, bs).astype(jnp.int32),
                   Y.reshape(B * (out_pad // 2)))

    active = act.reshape(B, n_active).astype(jnp.int64)
    out = vals.reshape(B, n_active)
    return (out, active)


# speedup vs baseline: 84.0964x; 1.0008x over previous
import functools

import jax
import jax.numpy as jnp
from jax import lax
from jax.experimental import pallas as pl
from jax.experimental.pallas import tpu as pltpu
from jax.experimental.pallas import tpu_sc as plsc


def _hash_into(x, proj_ref, row_ref):
    s = lax.dot_general(
        x, proj_ref[...], (((1,), (0,)), ((), ())),
        preferred_element_type=jnp.float32, precision=lax.Precision.DEFAULT)
    LK = proj_ref.shape[1]
    L = row_ref.shape[1]
    K = LK // L
    bits = (s > 0).astype(jnp.float32)
    jj = lax.broadcasted_iota(jnp.int32, (LK, L), 0)
    ll = lax.broadcasted_iota(jnp.int32, (LK, L), 1)
    amt = jj - K * ll
    sel = (amt >= 0) & (amt < K)
    M = jnp.where(sel, jnp.int32(1) << jnp.clip(amt, 0, K - 1), 0)
    h = lax.dot_general(
        bits, M.astype(jnp.float32), (((1,), (0,)), ((), ())),
        preferred_element_type=jnp.float32, precision=lax.Precision.HIGHEST)
    lrow = lax.broadcasted_iota(jnp.int32, h.shape, 1)
    row_ref[...] = h.astype(jnp.int32) + jnp.int32(2 ** K) * lrow


def _logits_body(x_ref, w_ref, b_ref, proj_ref, y_ref, row_ref):
    @pl.when(pl.program_id(0) == 0)
    def _():
        _hash_into(x_ref[...], proj_ref, row_ref)

    y = lax.dot_general(
        x_ref[...], w_ref[...], (((1,), (1,)), ((), ())),
        preferred_element_type=jnp.float32, precision=lax.Precision.DEFAULT)
    y = y + b_ref[...]
    bn = y.shape[1]
    u = lax.bitcast_convert_type(y, jnp.uint32)
    packed = (u[:, :bn // 2] >> 16) | (u[:, bn // 2:] & jnp.uint32(0xFFFF0000))
    y_ref[...] = packed.reshape(y_ref.shape)


def _sc_body(out_half, spw, row_hbm, bkt_hbm, y_hbm, act_hbm, val_hbm,
             idxs_v, cand_v, absidx_v, half_v, vals_v, vals_f, sem_a, sem_b):
    nc = 2
    wid = lax.axis_index("s") * nc + lax.axis_index("c")
    words = spw * 8
    base_w = wid * words
    n_seg = words // 128
    for j in range(n_seg):
        pltpu.sync_copy(row_hbm.at[pl.ds(base_w + j * 128, 128)],
                        idxs_v.at[j])
    bkt_handles = [
        pltpu.async_copy(bkt_hbm.at[idxs_v.at[j]],
                         cand_v.at[pl.ds(j * 128, 128)], sem_a)
        for j in range(n_seg)
    ]
    for hnd in bkt_handles:
        hnd.wait()
    act_handle = pltpu.async_copy(cand_v, act_hbm.at[wid], sem_a)

    base_s = wid * spw
    vrows = (spw * 512) // 128
    rows_per_sample = vrows // spw

    def absbody(r, carry):
        boff = (base_s + r // rows_per_sample) * out_half
        for h in range(8):
            sr = 2 * r + (h // 4)
            cc = (h % 4) * 16
            c = cand_v[sr, pl.ds(cc, 16)]
            absidx_v[r, pl.ds(h * 16, 16)] = (
                boff + ((c >> 11) << 10) + (c & 1023))
            cu = plsc.bitcast(c, jnp.uint32)
            half_v[r, pl.ds(h * 16, 16)] = (cu >> 10) & jnp.uint32(1)
        return carry

    lax.fori_loop(0, vrows, absbody, 0)

    def fire(r, carry):
        pltpu.async_copy(y_hbm.at[absidx_v.at[r]], vals_v.at[r], sem_b)
        return carry

    lax.fori_loop(0, vrows, fire, 0)

    def drain(r, carry):
        pltpu.make_async_copy(y_hbm.at[absidx_v.at[r]], vals_v.at[r],
                              sem_b).wait()
        for h in range(8):
            sl = pl.ds(h * 16, 16)
            w = vals_v[r, sl]
            sh = jnp.uint32(16) - (half_v[r, sl] << 4)
            t = (w << sh) & jnp.uint32(0xFFFF0000)
            vals_f[r, sl] = plsc.bitcast(t, jnp.float32)
        return carry

    lax.fori_loop(0, vrows, drain, 0)
    pltpu.sync_copy(vals_f, val_hbm.at[wid])
    act_handle.wait()


def kernel(in_values, active_out_indices, W, bias, proj, buckets):
    B, D = in_values.shape
    out_dim = W.shape[0]
    Lt, nbk, bs = buckets.shape
    n_active = active_out_indices.shape[1]

    bn = 2048
    n_blk = pl.cdiv(out_dim, bn)
    out_pad = n_blk * bn
    Y, rowflat = pl.pallas_call(
        _logits_body,
        grid=(n_blk,),
        in_specs=[
            pl.BlockSpec((B, D), lambda i: (0, 0)),
            pl.BlockSpec((bn, D), lambda i: (i, 0)),
            pl.BlockSpec((1, bn), lambda i: (0, i)),
            pl.BlockSpec((D, proj.shape[1]), lambda i: (0, 0)),
        ],
        out_specs=[
            pl.BlockSpec((B, bn // 256, 128), lambda i: (0, i, 0)),
            pl.BlockSpec((B, Lt), lambda i: (0, 0)),
        ],
        out_shape=[
            jax.ShapeDtypeStruct((B, out_pad // 256, 128), jnp.uint32),
            jax.ShapeDtypeStruct((B, Lt), jnp.int32),
        ],
    )(in_values, W, bias.reshape(1, out_dim), proj)

    info = plsc.get_sparse_core_info()
    nw = info.num_cores * info.num_subcores
    spw = B // nw
    words = spw * Lt
    vrows = (spw * Lt * bs) // 128

    mesh = plsc.VectorSubcoreMesh(core_axis_name="c", subcore_axis_name="s")
    sc = pl.kernel(
        functools.partial(_sc_body, out_pad // 2, spw),
        out_type=[
            jax.ShapeDtypeStruct((nw, words, bs), jnp.int32),
            jax.ShapeDtypeStruct((nw, vrows, 128), jnp.float32),
        ],
        mesh=mesh,
        compiler_params=pltpu.CompilerParams(use_tc_tiling_on_sc=False,
                                             needs_layout_passes=False),
        scratch_types=[
            pltpu.VMEM((words // 128, 128), jnp.int32),
            pltpu.VMEM((words, bs), jnp.int32),
            pltpu.VMEM((vrows, 128), jnp.int32),
            pltpu.VMEM((vrows, 128), jnp.uint32),
            pltpu.VMEM((vrows, 128), jnp.uint32),
            pltpu.VMEM((vrows, 128), jnp.float32),
            pltpu.SemaphoreType.DMA,
            pltpu.SemaphoreType.DMA,
        ],
    )
    act, vals = sc(rowflat.reshape(B * Lt),
                   buckets.reshape(Lt * nbk, bs).astype(jnp.int32),
                   Y.reshape(B * (out_pad // 2)))

    active = act.reshape(B, n_active).astype(jnp.int64)
    out = vals.reshape(B, n_active)
    return (out, active)


# speedup vs baseline: 84.5847x; 1.0058x over previous
import functools

import jax
import jax.numpy as jnp
from jax import lax
from jax.experimental import pallas as pl
from jax.experimental.pallas import tpu as pltpu
from jax.experimental.pallas import tpu_sc as plsc


def _hash_into(x, proj_ref, row_ref):
    s = lax.dot_general(
        x, proj_ref[...], (((1,), (0,)), ((), ())),
        preferred_element_type=jnp.float32, precision=lax.Precision.DEFAULT)
    LK = proj_ref.shape[1]
    L = row_ref.shape[1]
    K = LK // L
    bits = (s > 0).astype(jnp.float32)
    jj = lax.broadcasted_iota(jnp.int32, (LK, L), 0)
    ll = lax.broadcasted_iota(jnp.int32, (LK, L), 1)
    amt = jj - K * ll
    sel = (amt >= 0) & (amt < K)
    M = jnp.where(sel, jnp.int32(1) << jnp.clip(amt, 0, K - 1), 0)
    h = lax.dot_general(
        bits, M.astype(jnp.float32), (((1,), (0,)), ((), ())),
        preferred_element_type=jnp.float32, precision=lax.Precision.HIGHEST)
    lrow = lax.broadcasted_iota(jnp.int32, h.shape, 1)
    row_ref[...] = h.astype(jnp.int32) + jnp.int32(2 ** K) * lrow


def _logits_body(x_ref, w_ref, b_ref, proj_ref, y_ref, row_ref):
    @pl.when(pl.program_id(0) == 0)
    def _():
        _hash_into(x_ref[...], proj_ref, row_ref)

    y = lax.dot_general(
        x_ref[...], w_ref[...], (((1,), (1,)), ((), ())),
        preferred_element_type=jnp.float32, precision=lax.Precision.DEFAULT)
    y = y + b_ref[...]
    bn = y.shape[1]
    u = lax.bitcast_convert_type(y, jnp.uint32)
    packed = (u[:, :bn // 2] >> 16) | (u[:, bn // 2:] & jnp.uint32(0xFFFF0000))
    y_ref[...] = packed.reshape(y_ref.shape)


def _sc_body(out_half, bn, spw, row_hbm, bkt_hbm, y_hbm, act_hbm, val_hbm,
             idxs_v, cand_v, absidx_v, half_v, vals_v, vals_f, sem_a, sem_b):
    nc = 2
    wid = lax.axis_index("s") * nc + lax.axis_index("c")
    words = spw * 8
    base_w = wid * words
    n_seg = words // 128
    for j in range(n_seg):
        pltpu.sync_copy(row_hbm.at[pl.ds(base_w + j * 128, 128)],
                        idxs_v.at[j])
    bkt_handles = [
        pltpu.async_copy(bkt_hbm.at[idxs_v.at[j]],
                         cand_v.at[pl.ds(j * 128, 128)], sem_a)
        for j in range(n_seg)
    ]
    for hnd in bkt_handles:
        hnd.wait()
    act_handle = pltpu.async_copy(cand_v, act_hbm.at[wid], sem_a)

    base_s = wid * spw
    vrows = (spw * 512) // 128
    rows_per_sample = vrows // spw

    sb = bn.bit_length() - 1
    sh = sb - 1
    lowmask = (bn // 2) - 1

    def absbody(r, carry):
        boff = (base_s + r // rows_per_sample) * out_half
        for h in range(8):
            sr = 2 * r + (h // 4)
            cc = (h % 4) * 16
            c = cand_v[sr, pl.ds(cc, 16)]
            absidx_v[r, pl.ds(h * 16, 16)] = (
                boff + ((c >> sb) << sh) + (c & lowmask))
            cu = plsc.bitcast(c, jnp.uint32)
            half_v[r, pl.ds(h * 16, 16)] = (cu >> sh) & jnp.uint32(1)
        return carry

    lax.fori_loop(0, vrows, absbody, 0)

    def fire(r, carry):
        pltpu.async_copy(y_hbm.at[absidx_v.at[r]], vals_v.at[r], sem_b)
        return carry

    lax.fori_loop(0, vrows, fire, 0)

    def drain(r, carry):
        pltpu.make_async_copy(y_hbm.at[absidx_v.at[r]], vals_v.at[r],
                              sem_b).wait()
        for h in range(8):
            sl = pl.ds(h * 16, 16)
            w = vals_v[r, sl]
            sh = jnp.uint32(16) - (half_v[r, sl] << 4)
            t = (w << sh) & jnp.uint32(0xFFFF0000)
            vals_f[r, sl] = plsc.bitcast(t, jnp.float32)
        return carry

    lax.fori_loop(0, vrows, drain, 0)
    pltpu.sync_copy(vals_f, val_hbm.at[wid])
    act_handle.wait()


def kernel(in_values, active_out_indices, W, bias, proj, buckets):
    B, D = in_values.shape
    out_dim = W.shape[0]
    Lt, nbk, bs = buckets.shape
    n_active = active_out_indices.shape[1]

    bn = 4096
    n_blk = pl.cdiv(out_dim, bn)
    out_pad = n_blk * bn
    Y, rowflat = pl.pallas_call(
        _logits_body,
        grid=(n_blk,),
        in_specs=[
            pl.BlockSpec((B, D), lambda i: (0, 0)),
            pl.BlockSpec((bn, D), lambda i: (i, 0)),
            pl.BlockSpec((1, bn), lambda i: (0, i)),
            pl.BlockSpec((D, proj.shape[1]), lambda i: (0, 0)),
        ],
        out_specs=[
            pl.BlockSpec((B, bn // 256, 128), lambda i: (0, i, 0)),
            pl.BlockSpec((B, Lt), lambda i: (0, 0)),
        ],
        out_shape=[
            jax.ShapeDtypeStruct((B, out_pad // 256, 128), jnp.uint32),
            jax.ShapeDtypeStruct((B, Lt), jnp.int32),
        ],
    )(in_values, W, bias.reshape(1, out_dim), proj)

    info = plsc.get_sparse_core_info()
    nw = info.num_cores * info.num_subcores
    spw = B // nw
    words = spw * Lt
    vrows = (spw * Lt * bs) // 128

    mesh = plsc.VectorSubcoreMesh(core_axis_name="c", subcore_axis_name="s")
    sc = pl.kernel(
        functools.partial(_sc_body, out_pad // 2, bn, spw),
        out_type=[
            jax.ShapeDtypeStruct((nw, words, bs), jnp.int32),
            jax.ShapeDtypeStruct((nw, vrows, 128), jnp.float32),
        ],
        mesh=mesh,
        compiler_params=pltpu.CompilerParams(use_tc_tiling_on_sc=False,
                                             needs_layout_passes=False),
        scratch_types=[
            pltpu.VMEM((words // 128, 128), jnp.int32),
            pltpu.VMEM((words, bs), jnp.int32),
            pltpu.VMEM((vrows, 128), jnp.int32),
            pltpu.VMEM((vrows, 128), jnp.uint32),
            pltpu.VMEM((vrows, 128), jnp.uint32),
            pltpu.VMEM((vrows, 128), jnp.float32),
            pltpu.SemaphoreType.DMA,
            pltpu.SemaphoreType.DMA,
        ],
    )
    act, vals = sc(rowflat.reshape(B * Lt),
                   buckets.reshape(Lt * nbk, bs).astype(jnp.int32),
                   Y.reshape(B * (out_pad // 2)))

    active = act.reshape(B, n_active).astype(jnp.int64)
    out = vals.reshape(B, n_active)
    return (out, active)


# speedup vs baseline: 86.3841x; 1.0213x over previous
import functools

import jax
import jax.numpy as jnp
from jax import lax
from jax.experimental import pallas as pl
from jax.experimental.pallas import tpu as pltpu
from jax.experimental.pallas import tpu_sc as plsc


def _hash_into(x, proj_ref, row_ref):
    s = lax.dot_general(
        x, proj_ref[...], (((1,), (0,)), ((), ())),
        preferred_element_type=jnp.float32, precision=lax.Precision.DEFAULT)
    LK = proj_ref.shape[1]
    L = row_ref.shape[1]
    K = LK // L
    bits = (s > 0).astype(jnp.float32)
    jj = lax.broadcasted_iota(jnp.int32, (LK, L), 0)
    ll = lax.broadcasted_iota(jnp.int32, (LK, L), 1)
    amt = jj - K * ll
    sel = (amt >= 0) & (amt < K)
    M = jnp.where(sel, jnp.int32(1) << jnp.clip(amt, 0, K - 1), 0)
    h = lax.dot_general(
        bits, M.astype(jnp.float32), (((1,), (0,)), ((), ())),
        preferred_element_type=jnp.float32, precision=lax.Precision.HIGHEST)
    lrow = lax.broadcasted_iota(jnp.int32, h.shape, 1)
    row_ref[...] = h.astype(jnp.int32) + jnp.int32(2 ** K) * lrow


def _hash_body(x_ref, proj_ref, row_ref):
    _hash_into(x_ref[...], proj_ref, row_ref)


def _logits_body(x_ref, w_ref, b_ref, y_ref):
    y = lax.dot_general(
        x_ref[...], w_ref[...], (((1,), (1,)), ((), ())),
        preferred_element_type=jnp.float32, precision=lax.Precision.DEFAULT)
    y = y + b_ref[...]
    bn = y.shape[1]
    u = lax.bitcast_convert_type(y, jnp.uint32)
    packed = (u[:, :bn // 2] >> 16) | (u[:, bn // 2:] & jnp.uint32(0xFFFF0000))
    y_ref[...] = packed.reshape(y_ref.shape)


def _sc1_body(out_half, bn, spw, row_hbm, bkt_hbm, act_hbm, absidx_hbm,
              half_hbm, idxs_v, cand_v, absidx_v, half_v, sem_a):
    nc = 2
    wid = lax.axis_index("s") * nc + lax.axis_index("c")
    words = spw * 8
    base_w = wid * words
    n_seg = words // 128
    for j in range(n_seg):
        pltpu.sync_copy(row_hbm.at[pl.ds(base_w + j * 128, 128)],
                        idxs_v.at[j])
    bkt_handles = [
        pltpu.async_copy(bkt_hbm.at[idxs_v.at[j]],
                         cand_v.at[pl.ds(j * 128, 128)], sem_a)
        for j in range(n_seg)
    ]
    for hnd in bkt_handles:
        hnd.wait()
    act_handle = pltpu.async_copy(cand_v, act_hbm.at[wid], sem_a)

    base_s = wid * spw
    vrows = (spw * 512) // 128
    rows_per_sample = vrows // spw

    sb = bn.bit_length() - 1
    sh = sb - 1
    lowmask = (bn // 2) - 1

    def absbody(r, carry):
        boff = (base_s + r // rows_per_sample) * out_half
        for h in range(8):
            sr = 2 * r + (h // 4)
            cc = (h % 4) * 16
            c = cand_v[sr, pl.ds(cc, 16)]
            absidx_v[r, pl.ds(h * 16, 16)] = (
                boff + ((c >> sb) << sh) + (c & lowmask))
            cu = plsc.bitcast(c, jnp.uint32)
            half_v[r, pl.ds(h * 16, 16)] = (cu >> sh) & jnp.uint32(1)
        return carry

    lax.fori_loop(0, vrows, absbody, 0)
    pltpu.sync_copy(absidx_v, absidx_hbm.at[wid])
    pltpu.sync_copy(half_v, half_hbm.at[wid])
    act_handle.wait()


def _sc2_body(spw, absidx_hbm, half_hbm, y_hbm, val_hbm,
              absidx_v, half_v, vals_v, vals_f, sem_b):
    nc = 2
    wid = lax.axis_index("s") * nc + lax.axis_index("c")
    vrows = (spw * 512) // 128

    pltpu.sync_copy(absidx_hbm.at[wid], absidx_v)
    pltpu.sync_copy(half_hbm.at[wid], half_v)

    def fire(r, carry):
        pltpu.async_copy(y_hbm.at[absidx_v.at[r]], vals_v.at[r], sem_b)
        return carry

    lax.fori_loop(0, vrows, fire, 0)

    def drain(r, carry):
        pltpu.make_async_copy(y_hbm.at[absidx_v.at[r]], vals_v.at[r],
                              sem_b).wait()
        for h in range(8):
            sl = pl.ds(h * 16, 16)
            w = vals_v[r, sl]
            sh = jnp.uint32(16) - (half_v[r, sl] << 4)
            t = (w << sh) & jnp.uint32(0xFFFF0000)
            vals_f[r, sl] = plsc.bitcast(t, jnp.float32)
        return carry

    lax.fori_loop(0, vrows, drain, 0)
    pltpu.sync_copy(vals_f, val_hbm.at[wid])


def kernel(in_values, active_out_indices, W, bias, proj, buckets):
    B, D = in_values.shape
    out_dim = W.shape[0]
    Lt, nbk, bs = buckets.shape
    n_active = active_out_indices.shape[1]

    bn = 4096
    n_blk = pl.cdiv(out_dim, bn)
    out_pad = n_blk * bn

    rowflat = pl.pallas_call(
        _hash_body,
        out_shape=jax.ShapeDtypeStruct((B, Lt), jnp.int32),
    )(in_values, proj)

    info = plsc.get_sparse_core_info()
    nw = info.num_cores * info.num_subcores
    spw = B // nw
    words = spw * Lt
    vrows = (spw * Lt * bs) // 128
    mesh = plsc.VectorSubcoreMesh(core_axis_name="c", subcore_axis_name="s")
    sc_params = pltpu.CompilerParams(use_tc_tiling_on_sc=False,
                                     needs_layout_passes=False)

    sc1 = pl.kernel(
        functools.partial(_sc1_body, out_pad // 2, bn, spw),
        out_type=[
            jax.ShapeDtypeStruct((nw, words, bs), jnp.int32),
            jax.ShapeDtypeStruct((nw, vrows, 128), jnp.int32),
            jax.ShapeDtypeStruct((nw, vrows, 128), jnp.uint32),
        ],
        mesh=mesh,
        compiler_params=sc_params,
        scratch_types=[
            pltpu.VMEM((words // 128, 128), jnp.int32),
            pltpu.VMEM((words, bs), jnp.int32),
            pltpu.VMEM((vrows, 128), jnp.int32),
            pltpu.VMEM((vrows, 128), jnp.uint32),
            pltpu.SemaphoreType.DMA,
        ],
    )
    act, absidx_h, half_h = sc1(rowflat.reshape(B * Lt),
                                buckets.reshape(Lt * nbk, bs).astype(jnp.int32))

    Y = pl.pallas_call(
        _logits_body,
        grid=(n_blk,),
        in_specs=[
            pl.BlockSpec((B, D), lambda i: (0, 0)),
            pl.BlockSpec((bn, D), lambda i: (i, 0)),
            pl.BlockSpec((1, bn), lambda i: (0, i)),
        ],
        out_specs=pl.BlockSpec((B, bn // 256, 128), lambda i: (0, i, 0)),
        out_shape=jax.ShapeDtypeStruct((B, out_pad // 256, 128), jnp.uint32),
    )(in_values, W, bias.reshape(1, out_dim))

    sc2 = pl.kernel(
        functools.partial(_sc2_body, spw),
        out_type=jax.ShapeDtypeStruct((nw, vrows, 128), jnp.float32),
        mesh=mesh,
        compiler_params=sc_params,
        scratch_types=[
            pltpu.VMEM((vrows, 128), jnp.int32),
            pltpu.VMEM((vrows, 128), jnp.uint32),
            pltpu.VMEM((vrows, 128), jnp.uint32),
            pltpu.VMEM((vrows, 128), jnp.float32),
            pltpu.SemaphoreType.DMA,
        ],
    )
    vals = sc2(absidx_h, half_h, Y.reshape(B * (out_pad // 2)))

    active = act.reshape(B, n_active).astype(jnp.int64)
    out = vals.reshape(B, n_active)
    return (out, active)
